# Initial kernel scaffold; baseline (speedup 1.0000x reference)
import functools

import jax
import jax.numpy as jnp
from jax import lax
from jax.experimental import pallas as pl
from jax.experimental.pallas import tpu as pltpu
from jax.experimental.pallas import tpu_sc as plsc

DIM = 64
LANES = 16
CHUNK = 64
NEG_K = 5


def _sc_scores(u_idx, vp_idx, vn_idx_t, u_table, v_table):
    B = u_idx.shape[0]
    info = plsc.get_sparse_core_info()
    nc, ns = info.num_cores, info.num_subcores
    nw = nc * ns
    per_w = B // nw
    n_chunks = per_w // CHUNK
    mesh = plsc.VectorSubcoreMesh(core_axis_name="c", subcore_axis_name="s")

    @functools.partial(
        pl.kernel,
        mesh=mesh,
        out_type=(
            jax.ShapeDtypeStruct((B,), jnp.float32),
            jax.ShapeDtypeStruct((B,), jnp.float32),
        ),
        scratch_types=[
            pltpu.VMEM((per_w,), jnp.int32),
            pltpu.VMEM((per_w,), jnp.int32),
            pltpu.VMEM((NEG_K, per_w), jnp.int32),
            pltpu.VMEM((2, CHUNK, DIM), jnp.float32),
            pltpu.VMEM((2, CHUNK, DIM), jnp.float32),
            pltpu.VMEM((2, NEG_K, CHUNK, DIM), jnp.float32),
            pltpu.VMEM((LANES * CHUNK,), jnp.float32),
            pltpu.VMEM((LANES * CHUNK,), jnp.float32),
            pltpu.VMEM((per_w,), jnp.float32),
            pltpu.VMEM((per_w,), jnp.float32),
            pltpu.SemaphoreType.DMA,
            pltpu.SemaphoreType.DMA,
        ],
    )
    def k(u_idx_h, vp_idx_h, vn_idx_h, u_tab, v_tab, pos_out, neg_out,
          u_iv, vp_iv, vn_iv, u_b, vp_b, vn_b, pscr, nscr, pos_sv, neg_sv,
          sem0, sem1):
        wid = lax.axis_index("s") * nc + lax.axis_index("c")
        wbase = wid * per_w

        pltpu.sync_copy(u_idx_h.at[pl.ds(wbase, per_w)], u_iv)
        pltpu.sync_copy(vp_idx_h.at[pl.ds(wbase, per_w)], vp_iv)
        for n in range(NEG_K):
            pltpu.sync_copy(vn_idx_h.at[n, pl.ds(wbase, per_w)], vn_iv.at[n])

        sems = (sem0, sem1)

        def fire(c):
            p = c % 2
            cb = c * CHUNK
            hs = [
                pltpu.async_copy(
                    u_tab.at[u_iv.at[pl.ds(cb, CHUNK)]], u_b.at[p], sems[p]),
                pltpu.async_copy(
                    v_tab.at[vp_iv.at[pl.ds(cb, CHUNK)]], vp_b.at[p], sems[p]),
            ]
            for n in range(NEG_K):
                hs.append(pltpu.async_copy(
                    v_tab.at[vn_iv.at[n, pl.ds(cb, CHUNK)]],
                    vn_b.at[p, n], sems[p]))
            return hs

        pending = fire(0)
        iota16 = lax.iota(jnp.int32, LANES)

        for c in range(n_chunks):
            nxt = fire(c + 1) if c + 1 < n_chunks else []
            for h in pending:
                h.wait()
            pending = nxt
            p = c % 2

            def elem_body(e, carry):
                pos_p = jnp.zeros((LANES,), jnp.float32)
                neg_p = jnp.zeros((LANES,), jnp.float32)
                for j in range(DIM // LANES):
                    sl = pl.ds(j * LANES, LANES)
                    u = u_b[p, e, sl]
                    vsum = vn_b[p, 0, e, sl]
                    for n in range(1, NEG_K):
                        vsum = vsum + vn_b[p, n, e, sl]
                    pos_p = pos_p + u * vp_b[p, e, sl]
                    neg_p = neg_p + u * vsum
                sidx = iota16 * CHUNK + e
                plsc.store_scatter(pscr, [sidx], pos_p)
                plsc.store_scatter(nscr, [sidx], neg_p)
                return carry

            lax.fori_loop(0, CHUNK, elem_body, 0)

            def red_body(g, carry):
                gb = g * LANES
                acc_p = pscr[pl.ds(gb, LANES)]
                acc_n = nscr[pl.ds(gb, LANES)]
                for l in range(1, LANES):
                    acc_p = acc_p + pscr[pl.ds(l * CHUNK + gb, LANES)]
                    acc_n = acc_n + nscr[pl.ds(l * CHUNK + gb, LANES)]
                off = c * CHUNK + gb
                pos_sv[pl.ds(off, LANES)] = acc_p
                neg_sv[pl.ds(off, LANES)] = acc_n
                return carry

            lax.fori_loop(0, CHUNK // LANES, red_body, 0)

        pltpu.sync_copy(pos_sv, pos_out.at[pl.ds(wbase, per_w)])
        pltpu.sync_copy(neg_sv, neg_out.at[pl.ds(wbase, per_w)])

    return k(u_idx, vp_idx, vn_idx_t, u_table, v_table)


def _tc_loss_body(pos_ref, neg_ref, bs_ref, out_ref):
    pos = pos_ref[...]
    neg = neg_ref[...]
    ls = jax.nn.log_sigmoid(pos) + jax.nn.log_sigmoid(-neg)
    out_ref[0, 0] = -jnp.sum(ls) / bs_ref[0].astype(jnp.float32)


def kernel(u_positive, v_positive, v_negative, batch_size, u_table, v_table):
    B = u_positive.shape[0]
    u_idx = u_positive.astype(jnp.int32)
    vp_idx = v_positive.astype(jnp.int32)
    vn_idx_t = v_negative.astype(jnp.int32).T

    pos_s, neg_s = _sc_scores(u_idx, vp_idx, vn_idx_t, u_table, v_table)

    rows = B // 128
    bs = jnp.asarray(batch_size, jnp.int32).reshape((1,))
    loss = pl.pallas_call(
        _tc_loss_body,
        out_shape=jax.ShapeDtypeStruct((1, 1), jnp.float32),
        in_specs=[
            pl.BlockSpec(memory_space=pltpu.VMEM),
            pl.BlockSpec(memory_space=pltpu.VMEM),
            pl.BlockSpec(memory_space=pltpu.SMEM),
        ],
        out_specs=pl.BlockSpec(memory_space=pltpu.SMEM),
    )(pos_s.reshape((rows, 128)), neg_s.reshape((rows, 128)), bs)
    return loss[0, 0]


# speedup vs baseline: 1.7489x; 1.7489x over previous
import functools

import jax
import jax.numpy as jnp
from jax import lax
from jax.experimental import pallas as pl
from jax.experimental.pallas import tpu as pltpu
from jax.experimental.pallas import tpu_sc as plsc

DIM = 64
LANES = 16
CHUNK = 64
NEG_K = 5


def _sc_scores(u_idx, vp_idx, vn_idx_t, u_table, v_table):
    B = u_idx.shape[0]
    info = plsc.get_sparse_core_info()
    nc, ns = info.num_cores, info.num_subcores
    nw = nc * ns
    per_w = B // nw
    n_chunks = per_w // CHUNK
    mesh = plsc.VectorSubcoreMesh(core_axis_name="c", subcore_axis_name="s")

    @functools.partial(
        pl.kernel,
        mesh=mesh,
        compiler_params=pltpu.CompilerParams(
            needs_layout_passes=False, use_tc_tiling_on_sc=False),
        out_type=(
            jax.ShapeDtypeStruct((B,), jnp.float32),
            jax.ShapeDtypeStruct((B,), jnp.float32),
        ),
        scratch_types=[
            pltpu.VMEM((per_w,), jnp.int32),
            pltpu.VMEM((per_w,), jnp.int32),
            pltpu.VMEM((NEG_K * per_w,), jnp.int32),
            pltpu.VMEM((2, CHUNK, DIM), jnp.float32),
            pltpu.VMEM((2, CHUNK, DIM), jnp.float32),
            pltpu.VMEM((2, NEG_K, CHUNK, DIM), jnp.float32),
            pltpu.VMEM((LANES * CHUNK,), jnp.float32),
            pltpu.VMEM((LANES * CHUNK,), jnp.float32),
            pltpu.VMEM((per_w,), jnp.float32),
            pltpu.VMEM((per_w,), jnp.float32),
            pltpu.SemaphoreType.DMA,
            pltpu.SemaphoreType.DMA,
        ],
    )
    def k(u_idx_h, vp_idx_h, vn_idx_h, u_tab, v_tab, pos_out, neg_out,
          u_iv, vp_iv, vn_iv, u_b, vp_b, vn_b, pscr, nscr, pos_sv, neg_sv,
          sem0, sem1):
        wid = lax.axis_index("s") * nc + lax.axis_index("c")
        wbase = wid * per_w

        pltpu.sync_copy(u_idx_h.at[pl.ds(wbase, per_w)], u_iv)
        pltpu.sync_copy(vp_idx_h.at[pl.ds(wbase, per_w)], vp_iv)
        for n in range(NEG_K):
            pltpu.sync_copy(vn_idx_h.at[pl.ds(n * B + wbase, per_w)],
                            vn_iv.at[pl.ds(n * per_w, per_w)])

        sems = (sem0, sem1)

        def fire(c):
            p = c % 2
            cb = c * CHUNK
            hs = [
                pltpu.async_copy(
                    u_tab.at[u_iv.at[pl.ds(cb, CHUNK)]], u_b.at[p], sems[p]),
                pltpu.async_copy(
                    v_tab.at[vp_iv.at[pl.ds(cb, CHUNK)]], vp_b.at[p], sems[p]),
            ]
            for n in range(NEG_K):
                hs.append(pltpu.async_copy(
                    v_tab.at[vn_iv.at[pl.ds(n * per_w + cb, CHUNK)]],
                    vn_b.at[p, n], sems[p]))
            return hs

        pending = fire(0)
        iota16 = lax.iota(jnp.int32, LANES)

        for c in range(n_chunks):
            nxt = fire(c + 1) if c + 1 < n_chunks else []
            for h in pending:
                h.wait()
            pending = nxt
            p = c % 2

            def elem_body(e, carry):
                pos_p = jnp.zeros((LANES,), jnp.float32)
                neg_p = jnp.zeros((LANES,), jnp.float32)
                for j in range(DIM // LANES):
                    sl = pl.ds(j * LANES, LANES)
                    u = u_b[p, e, sl]
                    vsum = vn_b[p, 0, e, sl]
                    for n in range(1, NEG_K):
                        vsum = vsum + vn_b[p, n, e, sl]
                    pos_p = pos_p + u * vp_b[p, e, sl]
                    neg_p = neg_p + u * vsum
                sidx = iota16 * CHUNK + e
                plsc.store_scatter(pscr, [sidx], pos_p)
                plsc.store_scatter(nscr, [sidx], neg_p)
                return carry

            lax.fori_loop(0, CHUNK, elem_body, 0)

            def red_body(g, carry):
                gb = g * LANES
                acc_p = pscr[pl.ds(gb, LANES)]
                acc_n = nscr[pl.ds(gb, LANES)]
                for l in range(1, LANES):
                    acc_p = acc_p + pscr[pl.ds(l * CHUNK + gb, LANES)]
                    acc_n = acc_n + nscr[pl.ds(l * CHUNK + gb, LANES)]
                off = c * CHUNK + gb
                pos_sv[pl.ds(off, LANES)] = acc_p
                neg_sv[pl.ds(off, LANES)] = acc_n
                return carry

            lax.fori_loop(0, CHUNK // LANES, red_body, 0)

        pltpu.sync_copy(pos_sv, pos_out.at[pl.ds(wbase, per_w)])
        pltpu.sync_copy(neg_sv, neg_out.at[pl.ds(wbase, per_w)])

    return k(u_idx, vp_idx, vn_idx_t, u_table, v_table)


def _tc_loss_body(pos_ref, neg_ref, bs_ref, out_ref):
    pos = pos_ref[...]
    neg = neg_ref[...]
    ls = jax.nn.log_sigmoid(pos) + jax.nn.log_sigmoid(-neg)
    out_ref[0, 0] = -jnp.sum(ls) / bs_ref[0].astype(jnp.float32)


def kernel(u_positive, v_positive, v_negative, batch_size, u_table, v_table):
    B = u_positive.shape[0]
    u_idx = u_positive.astype(jnp.int32)
    vp_idx = v_positive.astype(jnp.int32)
    vn_idx_t = v_negative.astype(jnp.int32).T.reshape((-1,))

    pos_s, neg_s = _sc_scores(u_idx, vp_idx, vn_idx_t, u_table, v_table)

    rows = B // 128
    bs = jnp.asarray(batch_size, jnp.int32).reshape((1,))
    loss = pl.pallas_call(
        _tc_loss_body,
        out_shape=jax.ShapeDtypeStruct((1, 1), jnp.float32),
        in_specs=[
            pl.BlockSpec(memory_space=pltpu.VMEM),
            pl.BlockSpec(memory_space=pltpu.VMEM),
            pl.BlockSpec(memory_space=pltpu.SMEM),
        ],
        out_specs=pl.BlockSpec(memory_space=pltpu.SMEM),
    )(pos_s.reshape((rows, 128)), neg_s.reshape((rows, 128)), bs)
    return loss[0, 0]
